# baseline (speedup 1.0000x reference)
import functools
import math

import jax
import jax.numpy as jnp
from jax.experimental import pallas as pl
from jax.experimental.pallas import tpu as pltpu

D = 768
NH = 12
HD = 64
NL = 12
FF = 3072
T_FRAMES = 8
N_TOK = 197
T_PAD = 224
N_PATCH = 196
ROWS = T_FRAMES * T_PAD
MLP_CHUNK = 448

_BF = jnp.bfloat16
_NEG = -1e30

_O_LN1G, _O_LN1B = 0, D
_O_QKVB = 2 * D
_O_PROJB = 2 * D + 3 * D
_O_LN2G = _O_PROJB + D
_O_LN2B = _O_LN2G + D
_O_FC1B = _O_LN2B + D
_O_FC2B = _O_FC1B + FF
_P_LEN = _O_FC2B + D


def _gelu(x):
    return x * 0.5 * (1.0 + jax.lax.erf(x * (1.0 / math.sqrt(2.0))))


def _ln_f32(x, g, b, eps):
    m = jnp.mean(x, axis=-1, keepdims=True)
    xc = x - m
    v = jnp.mean(xc * xc, axis=-1, keepdims=True)
    return xc * jax.lax.rsqrt(v + eps) * g + b


def _embed_body(patch_ref, w_ref, beff_ref, pos_ref, cls_ref, pg_ref, pb_ref,
                out_ref):
    p = patch_ref[0].astype(_BF)
    w = w_ref[...].astype(_BF)
    e = jnp.dot(p, w, preferred_element_type=jnp.float32)
    e = e + beff_ref[...] + pos_ref[...]
    cls_row = cls_ref[...]
    pad = jnp.zeros((T_PAD - N_TOK, D), jnp.float32)
    x = jnp.concatenate([cls_row, e, pad], axis=0)
    out_ref[0] = _ln_f32(x, pg_ref[...], pb_ref[...], 1e-6)


def _mega_body(*refs):
    (x0_ref, pr_ref) = refs[0:2]
    wq_refs = refs[2:2 + NL]
    wp_refs = refs[2 + NL:2 + 2 * NL]
    w1_refs = refs[2 + 2 * NL:2 + 3 * NL]
    w2_refs = refs[2 + 3 * NL:2 + 4 * NL]
    (ng_ref, nb_ref, lg_ref, lb_ref, hw1_ref, hb1_ref, hw2_ref, hb2_ref,
     out_ref,
     X, qf, pf, f1, f2, qb, pb_, f1b, f2b,
     sem_x, sem_q, sem_p, sem_1, sem_2) = refs[2 + 4 * NL:]

    def cp(src, dst, sem):
        return pltpu.make_async_copy(src, dst, sem)

    cp(x0_ref, X, sem_x).start()
    cp(wq_refs[0], qf, sem_q).start()
    cp(wp_refs[0], pf, sem_p).start()
    cp(x0_ref, X, sem_x).wait()

    col = jax.lax.broadcasted_iota(jnp.int32, (T_PAD, T_PAD), 1)
    key_mask = col < N_TOK
    scale = 0.125

    def prow(l, a, n):
        return pr_ref[pl.ds(l, 1), a:a + n]

    def layer_body(l, carry):
        cp(wq_refs[0], qf, sem_q).wait()
        cp(wp_refs[0], pf, sem_p).wait()
        qb[...] = qf[...].astype(_BF)
        pb_[...] = pf[...].astype(_BF)
        for j in range(NL):
            @pl.when(l == j)
            def _start_mlp():
                cp(w1_refs[j], f1, sem_1).start()
                cp(w2_refs[j], f2, sem_2).start()

        g1 = prow(l, _O_LN1G, D)
        o1 = prow(l, _O_LN1B, D)
        bq = prow(l, _O_QKVB, 3 * D)
        bp = prow(l, _O_PROJB, D)

        def pair_body(p):
            r0 = p * 2 * T_PAD
            x = X[pl.ds(r0, 2 * T_PAD), :]
            h = _ln_f32(x, g1, o1, 1e-6).astype(_BF)
            qkv = jnp.dot(h, qb[...],
                          preferred_element_type=jnp.float32) + bq
            qkvb = qkv.astype(_BF)
            halves = []
            for sub in range(2):
                rs = sub * T_PAD
                outs = []
                for hd in range(NH):
                    q = qkvb[rs:rs + T_PAD, hd * HD:(hd + 1) * HD] * scale
                    k = qkvb[rs:rs + T_PAD, D + hd * HD:D + (hd + 1) * HD]
                    v = qkvb[rs:rs + T_PAD,
                             2 * D + hd * HD:2 * D + (hd + 1) * HD]
                    s = jax.lax.dot_general(q, k, (((1,), (1,)), ((), ())),
                                            preferred_element_type=jnp.float32)
                    s = jnp.where(key_mask, s, _NEG)
                    s = s - jnp.max(s, axis=-1, keepdims=True)
                    e = jnp.exp(s)
                    a = (e / jnp.sum(e, axis=-1, keepdims=True)).astype(_BF)
                    outs.append(jnp.dot(a, v,
                                        preferred_element_type=jnp.float32))
                halves.append(jnp.concatenate(outs, axis=1).astype(_BF))
            o = jnp.concatenate(halves, axis=0)
            X[pl.ds(r0, 2 * T_PAD), :] = x + jnp.dot(
                o, pb_[...], preferred_element_type=jnp.float32) + bp

        for p in range(T_FRAMES // 2):
            pair_body(p)

        cp(w1_refs[0], f1, sem_1).wait()
        cp(w2_refs[0], f2, sem_2).wait()
        f1b[...] = f1[...].astype(_BF)
        f2b[...] = f2[...].astype(_BF)
        for j in range(NL - 1):
            @pl.when(l == j)
            def _start_attn():
                cp(wq_refs[j + 1], qf, sem_q).start()
                cp(wp_refs[j + 1], pf, sem_p).start()

        g2 = prow(l, _O_LN2G, D)
        o2 = prow(l, _O_LN2B, D)
        b1 = prow(l, _O_FC1B, FF)
        b2 = prow(l, _O_FC2B, D)

        def mlp_body(c):
            r0 = c * MLP_CHUNK
            x = X[pl.ds(r0, MLP_CHUNK), :]
            h = _ln_f32(x, g2, o2, 1e-6).astype(_BF)
            h1 = jnp.dot(h, f1b[...],
                         preferred_element_type=jnp.float32) + b1
            h1 = _gelu(h1).astype(_BF)
            X[pl.ds(r0, MLP_CHUNK), :] = x + jnp.dot(
                h1, f2b[...], preferred_element_type=jnp.float32) + b2

        for c in range(ROWS // MLP_CHUNK):
            mlp_body(c)
        return carry

    jax.lax.fori_loop(0, NL, layer_body, 0)

    cls_rows = jnp.concatenate(
        [X[f * T_PAD:f * T_PAD + 1, :] for f in range(T_FRAMES)], axis=0)
    feats = _ln_f32(cls_rows, ng_ref[...], nb_ref[...], 1e-6)
    mean_feat = jnp.mean(feats, axis=0, keepdims=True)
    dc = feats - mean_feat
    d = jnp.sum(dc * dc, axis=-1, keepdims=True)
    drow = d.reshape(1, T_FRAMES)
    idx = jax.lax.broadcasted_iota(jnp.int32, (1, T_FRAMES), 1)
    m1 = jnp.max(drow)
    i1 = jnp.min(jnp.where(drow == m1, idx, T_FRAMES))
    sel1 = idx == i1
    d2 = jnp.where(sel1, _NEG, drow)
    m2 = jnp.max(d2)
    i2 = jnp.min(jnp.where(d2 == m2, idx, T_FRAMES))
    keep = jnp.logical_not(jnp.logical_or(sel1, idx == i2))
    keep_f = keep.astype(jnp.float32)
    cnt = jnp.maximum(jnp.sum(keep_f), 1.0)
    pooled = jnp.sum(feats * keep_f.reshape(T_FRAMES, 1), axis=0,
                     keepdims=True) / cnt
    h = _ln_f32(pooled, lg_ref[...], lb_ref[...], 1e-5)
    h = jnp.dot(h.astype(_BF), hw1_ref[...].astype(_BF),
                preferred_element_type=jnp.float32) + hb1_ref[...]
    h = _gelu(h)
    out = jnp.dot(h.astype(_BF), hw2_ref[...].astype(_BF),
                  preferred_element_type=jnp.float32) + hb2_ref[...]
    out_ref[...] = out


def _row(v):
    return v.reshape(1, -1)


@functools.partial(jax.jit, static_argnames=("interpret",))
def kernel(x, params, interpret=False):
    B, T = x.shape[0], x.shape[1]
    frames = x.reshape(B * T, 3, 224, 224)
    patches = frames.reshape(B * T, 3, 14, 16, 14, 16)
    patches = patches.transpose(0, 2, 4, 1, 3, 5).reshape(B * T, N_PATCH, 768)

    mean = jnp.array([0.485, 0.456, 0.406], jnp.float32)
    std = jnp.array([0.229, 0.224, 0.225], jnp.float32)
    scale = (1.0 / (255.0 * std)).repeat(256)
    offset = (mean / std).repeat(256)
    w_eff = params['patch_w'] * scale[:, None]
    b_eff = params['patch_b'] - offset @ params['patch_w']
    pos = params['pos'][0]
    cls_row = params['cls'][0] + pos[:1]

    vspec = pl.BlockSpec(memory_space=pltpu.MemorySpace.VMEM)
    aspec = pl.BlockSpec(memory_space=pl.ANY)

    x0 = pl.pallas_call(
        _embed_body,
        grid=(B * T,),
        in_specs=[
            pl.BlockSpec((1, N_PATCH, 768), lambda f: (f, 0, 0)),
            vspec, vspec, vspec, vspec, vspec, vspec,
        ],
        out_specs=pl.BlockSpec((1, T_PAD, D), lambda f: (f, 0, 0)),
        out_shape=jax.ShapeDtypeStruct((B * T, T_PAD, D), jnp.float32),
        interpret=interpret,
    )(patches, w_eff, _row(b_eff), pos[1:], cls_row,
      _row(params['pre_g']), _row(params['pre_b']))
    x0 = x0.reshape(ROWS, D)

    blocks = params['blocks']
    pr = jnp.concatenate([
        jnp.concatenate([
            blk['ln1_g'], blk['ln1_b'], blk['qkv_b'], blk['proj_b'],
            blk['ln2_g'], blk['ln2_b'], blk['fc1_b'], blk['fc2_b'],
        ]).reshape(1, _P_LEN) for blk in blocks], axis=0)

    out = pl.pallas_call(
        _mega_body,
        in_specs=[aspec, vspec] + [aspec] * (4 * NL) + [vspec] * 8,
        out_specs=vspec,
        out_shape=jax.ShapeDtypeStruct((B, 2), jnp.float32),
        scratch_shapes=[
            pltpu.VMEM((ROWS, D), jnp.float32),
            pltpu.VMEM((D, 3 * D), jnp.float32),
            pltpu.VMEM((D, D), jnp.float32),
            pltpu.VMEM((D, FF), jnp.float32),
            pltpu.VMEM((FF, D), jnp.float32),
            pltpu.VMEM((D, 3 * D), _BF),
            pltpu.VMEM((D, D), _BF),
            pltpu.VMEM((D, FF), _BF),
            pltpu.VMEM((FF, D), _BF),
            pltpu.SemaphoreType.DMA,
            pltpu.SemaphoreType.DMA,
            pltpu.SemaphoreType.DMA,
            pltpu.SemaphoreType.DMA,
            pltpu.SemaphoreType.DMA,
        ],
        interpret=interpret,
    )(x0, pr,
      *[blk['qkv_w'] for blk in blocks],
      *[blk['proj_w'] for blk in blocks],
      *[blk['fc1_w'] for blk in blocks],
      *[blk['fc2_w'] for blk in blocks],
      _row(params['norm_g']), _row(params['norm_b']),
      _row(params['cls_ln_g']), _row(params['cls_ln_b']),
      params['head1_w'], _row(params['head1_b']),
      params['head2_w'], _row(params['head2_b']))
    return out


# speedup vs baseline: 1.0238x; 1.0238x over previous
import functools
import math

import jax
import jax.numpy as jnp
from jax.experimental import pallas as pl
from jax.experimental.pallas import tpu as pltpu

D = 768
NH = 12
HD = 64
NL = 12
FF = 3072
T_FRAMES = 8
N_TOK = 197
T_PAD = 224
N_PATCH = 196
ROWS = T_FRAMES * T_PAD
MLP_CHUNK = 448

_BF = jnp.bfloat16
_NEG = -1e30

_O_LN1G, _O_LN1B = 0, D
_O_QKVB = 2 * D
_O_PROJB = 2 * D + 3 * D
_O_LN2G = _O_PROJB + D
_O_LN2B = _O_LN2G + D
_O_FC1B = _O_LN2B + D
_O_FC2B = _O_FC1B + FF
_P_LEN = _O_FC2B + D


def _gelu(x):
    return x * 0.5 * (1.0 + jax.lax.erf(x * (1.0 / math.sqrt(2.0))))


def _ln_f32(x, g, b, eps):
    m = jnp.mean(x, axis=-1, keepdims=True)
    xc = x - m
    v = jnp.mean(xc * xc, axis=-1, keepdims=True)
    return xc * jax.lax.rsqrt(v + eps) * g + b


def _embed_body(patch_ref, w_ref, beff_ref, pos_ref, cls_ref, pg_ref, pb_ref,
                out_ref):
    p = patch_ref[0].astype(_BF)
    w = w_ref[...].astype(_BF)
    e = jnp.dot(p, w, preferred_element_type=jnp.float32)
    e = e + beff_ref[...] + pos_ref[...]
    cls_row = cls_ref[...]
    pad = jnp.zeros((T_PAD - N_TOK, D), jnp.float32)
    x = jnp.concatenate([cls_row, e, pad], axis=0)
    out_ref[0] = _ln_f32(x, pg_ref[...], pb_ref[...], 1e-6)


def _mega_body(*refs):
    (x0_ref, pr_ref) = refs[0:2]
    wq_refs = refs[2:2 + NL]
    wp_refs = refs[2 + NL:2 + 2 * NL]
    w1_refs = refs[2 + 2 * NL:2 + 3 * NL]
    w2_refs = refs[2 + 3 * NL:2 + 4 * NL]
    (ng_ref, nb_ref, lg_ref, lb_ref, hw1_ref, hb1_ref, hw2_ref, hb2_ref,
     out_ref,
     X, qf, pf, f1, f2, qb, pb_, f1b, f2b,
     sem_x, sem_q, sem_p, sem_1, sem_2) = refs[2 + 4 * NL:]

    def cp(src, dst, sem):
        return pltpu.make_async_copy(src, dst, sem)

    cp(x0_ref, X, sem_x).start()
    cp(wq_refs[0], qf, sem_q).start()
    cp(wp_refs[0], pf, sem_p).start()
    cp(x0_ref, X, sem_x).wait()

    col = jax.lax.broadcasted_iota(jnp.int32, (T_PAD, T_PAD), 1)
    key_mask = col < N_TOK
    scale = 0.125

    def prow(l, a, n):
        return pr_ref[pl.ds(l, 1), a:a + n]

    def layer_body(l, carry):
        cp(wq_refs[0], qf, sem_q).wait()
        cp(wp_refs[0], pf, sem_p).wait()
        qb[...] = qf[...].astype(_BF)
        pb_[...] = pf[...].astype(_BF)
        for j in range(NL):
            @pl.when(l == j)
            def _start_mlp():
                cp(w1_refs[j], f1, sem_1).start()
                cp(w2_refs[j], f2, sem_2).start()

        g1 = prow(l, _O_LN1G, D)
        o1 = prow(l, _O_LN1B, D)
        bq = prow(l, _O_QKVB, 3 * D)
        bp = prow(l, _O_PROJB, D)

        def frame_body(f):
            r0 = f * T_PAD
            x = X[pl.ds(r0, T_PAD), :]
            h = _ln_f32(x, g1, o1, 1e-6).astype(_BF)
            qkv = jnp.dot(h, qb[...],
                          preferred_element_type=jnp.float32) + bq
            qkvb = qkv.astype(_BF)
            outs = []
            for hd in range(NH):
                q = qkvb[:, hd * HD:(hd + 1) * HD] * scale
                k = qkvb[:, D + hd * HD:D + (hd + 1) * HD]
                v = qkvb[:, 2 * D + hd * HD:2 * D + (hd + 1) * HD]
                s = jax.lax.dot_general(q, k, (((1,), (1,)), ((), ())),
                                        preferred_element_type=jnp.float32)
                s = jnp.where(key_mask, s, _NEG)
                s = s - jnp.max(s, axis=-1, keepdims=True)
                e = jnp.exp(s)
                a = (e / jnp.sum(e, axis=-1, keepdims=True)).astype(_BF)
                outs.append(jnp.dot(a, v, preferred_element_type=jnp.float32))
            o = jnp.concatenate(outs, axis=1).astype(_BF)
            X[pl.ds(r0, T_PAD), :] = x + jnp.dot(
                o, pb_[...], preferred_element_type=jnp.float32) + bp

        for f in range(T_FRAMES):
            frame_body(f)

        cp(w1_refs[0], f1, sem_1).wait()
        cp(w2_refs[0], f2, sem_2).wait()
        f1b[...] = f1[...].astype(_BF)
        f2b[...] = f2[...].astype(_BF)
        for j in range(NL - 1):
            @pl.when(l == j)
            def _start_attn():
                cp(wq_refs[j + 1], qf, sem_q).start()
                cp(wp_refs[j + 1], pf, sem_p).start()

        g2 = prow(l, _O_LN2G, D)
        o2 = prow(l, _O_LN2B, D)
        b1 = prow(l, _O_FC1B, FF)
        b2 = prow(l, _O_FC2B, D)

        def mlp_body(c):
            r0 = c * MLP_CHUNK
            x = X[pl.ds(r0, MLP_CHUNK), :]
            h = _ln_f32(x, g2, o2, 1e-6).astype(_BF)
            h1 = jnp.dot(h, f1b[...],
                         preferred_element_type=jnp.float32) + b1
            h1 = _gelu(h1).astype(_BF)
            X[pl.ds(r0, MLP_CHUNK), :] = x + jnp.dot(
                h1, f2b[...], preferred_element_type=jnp.float32) + b2

        for c in range(ROWS // MLP_CHUNK):
            mlp_body(c)
        return carry

    jax.lax.fori_loop(0, NL, layer_body, 0)

    cls_rows = jnp.concatenate(
        [X[f * T_PAD:f * T_PAD + 1, :] for f in range(T_FRAMES)], axis=0)
    feats = _ln_f32(cls_rows, ng_ref[...], nb_ref[...], 1e-6)
    mean_feat = jnp.mean(feats, axis=0, keepdims=True)
    dc = feats - mean_feat
    d = jnp.sum(dc * dc, axis=-1, keepdims=True)
    drow = d.reshape(1, T_FRAMES)
    idx = jax.lax.broadcasted_iota(jnp.int32, (1, T_FRAMES), 1)
    m1 = jnp.max(drow)
    i1 = jnp.min(jnp.where(drow == m1, idx, T_FRAMES))
    sel1 = idx == i1
    d2 = jnp.where(sel1, _NEG, drow)
    m2 = jnp.max(d2)
    i2 = jnp.min(jnp.where(d2 == m2, idx, T_FRAMES))
    keep = jnp.logical_not(jnp.logical_or(sel1, idx == i2))
    keep_f = keep.astype(jnp.float32)
    cnt = jnp.maximum(jnp.sum(keep_f), 1.0)
    pooled = jnp.sum(feats * keep_f.reshape(T_FRAMES, 1), axis=0,
                     keepdims=True) / cnt
    h = _ln_f32(pooled, lg_ref[...], lb_ref[...], 1e-5)
    h = jnp.dot(h.astype(_BF), hw1_ref[...].astype(_BF),
                preferred_element_type=jnp.float32) + hb1_ref[...]
    h = _gelu(h)
    out = jnp.dot(h.astype(_BF), hw2_ref[...].astype(_BF),
                  preferred_element_type=jnp.float32) + hb2_ref[...]
    out_ref[...] = out


def _row(v):
    return v.reshape(1, -1)


@functools.partial(jax.jit, static_argnames=("interpret",))
def kernel(x, params, interpret=False):
    B, T = x.shape[0], x.shape[1]
    frames = x.reshape(B * T, 3, 224, 224)
    patches = frames.reshape(B * T, 3, 14, 16, 14, 16)
    patches = patches.transpose(0, 2, 4, 1, 3, 5).reshape(B * T, N_PATCH, 768)

    mean = jnp.array([0.485, 0.456, 0.406], jnp.float32)
    std = jnp.array([0.229, 0.224, 0.225], jnp.float32)
    scale = (1.0 / (255.0 * std)).repeat(256)
    offset = (mean / std).repeat(256)
    w_eff = params['patch_w'] * scale[:, None]
    b_eff = params['patch_b'] - offset @ params['patch_w']
    pos = params['pos'][0]
    cls_row = params['cls'][0] + pos[:1]

    vspec = pl.BlockSpec(memory_space=pltpu.MemorySpace.VMEM)
    aspec = pl.BlockSpec(memory_space=pl.ANY)

    x0 = pl.pallas_call(
        _embed_body,
        grid=(B * T,),
        in_specs=[
            pl.BlockSpec((1, N_PATCH, 768), lambda f: (f, 0, 0)),
            vspec, vspec, vspec, vspec, vspec, vspec,
        ],
        out_specs=pl.BlockSpec((1, T_PAD, D), lambda f: (f, 0, 0)),
        out_shape=jax.ShapeDtypeStruct((B * T, T_PAD, D), jnp.float32),
        interpret=interpret,
    )(patches, w_eff, _row(b_eff), pos[1:], cls_row,
      _row(params['pre_g']), _row(params['pre_b']))
    x0 = x0.reshape(ROWS, D)

    blocks = params['blocks']
    pr = jnp.concatenate([
        jnp.concatenate([
            blk['ln1_g'], blk['ln1_b'], blk['qkv_b'], blk['proj_b'],
            blk['ln2_g'], blk['ln2_b'], blk['fc1_b'], blk['fc2_b'],
        ]).reshape(1, _P_LEN) for blk in blocks], axis=0)

    out = pl.pallas_call(
        _mega_body,
        in_specs=[aspec, vspec] + [aspec] * (4 * NL) + [vspec] * 8,
        out_specs=vspec,
        out_shape=jax.ShapeDtypeStruct((B, 2), jnp.float32),
        scratch_shapes=[
            pltpu.VMEM((ROWS, D), jnp.float32),
            pltpu.VMEM((D, 3 * D), jnp.float32),
            pltpu.VMEM((D, D), jnp.float32),
            pltpu.VMEM((D, FF), jnp.float32),
            pltpu.VMEM((FF, D), jnp.float32),
            pltpu.VMEM((D, 3 * D), _BF),
            pltpu.VMEM((D, D), _BF),
            pltpu.VMEM((D, FF), _BF),
            pltpu.VMEM((FF, D), _BF),
            pltpu.SemaphoreType.DMA,
            pltpu.SemaphoreType.DMA,
            pltpu.SemaphoreType.DMA,
            pltpu.SemaphoreType.DMA,
            pltpu.SemaphoreType.DMA,
        ],
        interpret=interpret,
    )(x0, pr,
      *[blk['qkv_w'] for blk in blocks],
      *[blk['proj_w'] for blk in blocks],
      *[blk['fc1_w'] for blk in blocks],
      *[blk['fc2_w'] for blk in blocks],
      _row(params['norm_g']), _row(params['norm_b']),
      _row(params['cls_ln_g']), _row(params['cls_ln_b']),
      params['head1_w'], _row(params['head1_b']),
      params['head2_w'], _row(params['head2_b']))
    return out


# speedup vs baseline: 1.0646x; 1.0398x over previous
import functools
import math

import jax
import jax.numpy as jnp
from jax.experimental import pallas as pl
from jax.experimental.pallas import tpu as pltpu

D = 768
NH = 12
HD = 64
NL = 12
FF = 3072
T_FRAMES = 8
N_TOK = 197
T_PAD = 224
N_PATCH = 196
ROWS = T_FRAMES * T_PAD
MLP_CHUNK = 448

_BF = jnp.bfloat16
_NEG = -1e30

_O_LN1G, _O_LN1B = 0, D
_O_QKVB = 2 * D
_O_PROJB = 2 * D + 3 * D
_O_LN2G = _O_PROJB + D
_O_LN2B = _O_LN2G + D
_O_FC1B = _O_LN2B + D
_O_FC2B = _O_FC1B + FF
_P_LEN = _O_FC2B + D


def _gelu(x):
    return x * 0.5 * (1.0 + jax.lax.erf(x * (1.0 / math.sqrt(2.0))))


def _ln_f32(x, g, b, eps):
    m = jnp.mean(x, axis=-1, keepdims=True)
    xc = x - m
    v = jnp.mean(xc * xc, axis=-1, keepdims=True)
    return xc * jax.lax.rsqrt(v + eps) * g + b


def _mega_body(*refs):
    pr_ref = refs[0]
    wq_refs = refs[1:1 + NL]
    wp_refs = refs[1 + NL:1 + 2 * NL]
    w1_refs = refs[1 + 2 * NL:1 + 3 * NL]
    w2_refs = refs[1 + 3 * NL:1 + 4 * NL]
    (pp_ref, we_ref, posb_ref, cls_ref, pg_ref, pb0_ref,
     ng_ref, nb_ref, lg_ref, lb_ref, hw1_ref, hb1_ref, hw2_ref, hb2_ref,
     out_ref,
     X, qf, pf, f1, f2, qb, pb_, f1b, f2b,
     sem_q, sem_p, sem_1, sem_2) = refs[1 + 4 * NL:]

    def cp(src, dst, sem):
        return pltpu.make_async_copy(src, dst, sem)

    cp(wq_refs[0], qf, sem_q).start()
    cp(wp_refs[0], pf, sem_p).start()

    we = we_ref[...]
    posb = posb_ref[...]
    cls_row = cls_ref[...]
    pad = jnp.zeros((T_PAD - N_TOK, D), jnp.float32)
    for f in range(T_FRAMES):
        p = pp_ref[f]
        e = jnp.dot(p, we, preferred_element_type=jnp.float32) + posb
        rows = jnp.concatenate([cls_row, e, pad], axis=0)
        X[pl.ds(f * T_PAD, T_PAD), :] = _ln_f32(
            rows, pg_ref[...], pb0_ref[...], 1e-6)

    col = jax.lax.broadcasted_iota(jnp.int32, (T_PAD, T_PAD), 1)
    key_mask = col < N_TOK
    scale = 0.125

    def prow(l, a, n):
        return pr_ref[pl.ds(l, 1), a:a + n]

    def layer_body(l, carry):
        cp(wq_refs[0], qf, sem_q).wait()
        cp(wp_refs[0], pf, sem_p).wait()
        qb[...] = qf[...].astype(_BF)
        pb_[...] = pf[...].astype(_BF)
        for j in range(NL):
            @pl.when(l == j)
            def _start_mlp():
                cp(w1_refs[j], f1, sem_1).start()
                cp(w2_refs[j], f2, sem_2).start()

        g1 = prow(l, _O_LN1G, D)
        o1 = prow(l, _O_LN1B, D)
        bq = prow(l, _O_QKVB, 3 * D)
        bp = prow(l, _O_PROJB, D)

        def frame_body(f):
            r0 = f * T_PAD
            x = X[pl.ds(r0, T_PAD), :]
            h = _ln_f32(x, g1, o1, 1e-6).astype(_BF)
            qkv = jnp.dot(h, qb[...],
                          preferred_element_type=jnp.float32) + bq
            qkvb = qkv.astype(_BF)
            outs = []
            for hd in range(NH):
                q = qkvb[:, hd * HD:(hd + 1) * HD] * scale
                k = qkvb[:, D + hd * HD:D + (hd + 1) * HD]
                v = qkvb[:, 2 * D + hd * HD:2 * D + (hd + 1) * HD]
                s = jax.lax.dot_general(q, k, (((1,), (1,)), ((), ())),
                                        preferred_element_type=jnp.float32)
                s = jnp.where(key_mask, s, _NEG)
                s = s - jnp.max(s, axis=-1, keepdims=True)
                e = jnp.exp(s)
                a = (e / jnp.sum(e, axis=-1, keepdims=True)).astype(_BF)
                outs.append(jnp.dot(a, v, preferred_element_type=jnp.float32))
            o = jnp.concatenate(outs, axis=1).astype(_BF)
            X[pl.ds(r0, T_PAD), :] = x + jnp.dot(
                o, pb_[...], preferred_element_type=jnp.float32) + bp

        for f in range(T_FRAMES):
            frame_body(f)

        cp(w1_refs[0], f1, sem_1).wait()
        cp(w2_refs[0], f2, sem_2).wait()
        f1b[...] = f1[...].astype(_BF)
        f2b[...] = f2[...].astype(_BF)
        for j in range(NL - 1):
            @pl.when(l == j)
            def _start_attn():
                cp(wq_refs[j + 1], qf, sem_q).start()
                cp(wp_refs[j + 1], pf, sem_p).start()

        g2 = prow(l, _O_LN2G, D)
        o2 = prow(l, _O_LN2B, D)
        b1 = prow(l, _O_FC1B, FF)
        b2 = prow(l, _O_FC2B, D)

        def mlp_body(c):
            r0 = c * MLP_CHUNK
            x = X[pl.ds(r0, MLP_CHUNK), :]
            h = _ln_f32(x, g2, o2, 1e-6).astype(_BF)
            h1 = jnp.dot(h, f1b[...],
                         preferred_element_type=jnp.float32) + b1
            h1 = _gelu(h1).astype(_BF)
            X[pl.ds(r0, MLP_CHUNK), :] = x + jnp.dot(
                h1, f2b[...], preferred_element_type=jnp.float32) + b2

        for c in range(ROWS // MLP_CHUNK):
            mlp_body(c)
        return carry

    jax.lax.fori_loop(0, NL, layer_body, 0)

    cls_rows = jnp.concatenate(
        [X[f * T_PAD:f * T_PAD + 1, :] for f in range(T_FRAMES)], axis=0)
    feats = _ln_f32(cls_rows, ng_ref[...], nb_ref[...], 1e-6)
    mean_feat = jnp.mean(feats, axis=0, keepdims=True)
    dc = feats - mean_feat
    d = jnp.sum(dc * dc, axis=-1, keepdims=True)
    drow = d.reshape(1, T_FRAMES)
    idx = jax.lax.broadcasted_iota(jnp.int32, (1, T_FRAMES), 1)
    m1 = jnp.max(drow)
    i1 = jnp.min(jnp.where(drow == m1, idx, T_FRAMES))
    sel1 = idx == i1
    d2 = jnp.where(sel1, _NEG, drow)
    m2 = jnp.max(d2)
    i2 = jnp.min(jnp.where(d2 == m2, idx, T_FRAMES))
    keep = jnp.logical_not(jnp.logical_or(sel1, idx == i2))
    keep_f = keep.astype(jnp.float32)
    cnt = jnp.maximum(jnp.sum(keep_f), 1.0)
    pooled = jnp.sum(feats * keep_f.reshape(T_FRAMES, 1), axis=0,
                     keepdims=True) / cnt
    h = _ln_f32(pooled, lg_ref[...], lb_ref[...], 1e-5)
    h = jnp.dot(h.astype(_BF), hw1_ref[...].astype(_BF),
                preferred_element_type=jnp.float32) + hb1_ref[...]
    h = _gelu(h)
    out = jnp.dot(h.astype(_BF), hw2_ref[...].astype(_BF),
                  preferred_element_type=jnp.float32) + hb2_ref[...]
    out_ref[...] = out


def _row(v):
    return v.reshape(1, -1)


@functools.partial(jax.jit, static_argnames=("interpret",))
def kernel(x, params, interpret=False):
    B, T = x.shape[0], x.shape[1]
    frames = x.reshape(B * T, 3, 224, 224)
    patches = frames.reshape(B * T, 3, 14, 16, 14, 16)
    patches = patches.transpose(0, 2, 4, 1, 3, 5).reshape(B * T, N_PATCH, 768)

    mean = jnp.array([0.485, 0.456, 0.406], jnp.float32)
    std = jnp.array([0.229, 0.224, 0.225], jnp.float32)
    scale = (1.0 / (255.0 * std)).repeat(256)
    offset = (mean / std).repeat(256)
    w_eff = params['patch_w'] * scale[:, None]
    b_eff = params['patch_b'] - offset @ params['patch_w']
    pos = params['pos'][0]
    posb = pos[1:] + b_eff
    cls_row = params['cls'][0] + pos[:1]

    vspec = pl.BlockSpec(memory_space=pltpu.MemorySpace.VMEM)
    aspec = pl.BlockSpec(memory_space=pl.ANY)

    blocks = params['blocks']
    pr = jnp.concatenate([
        jnp.concatenate([
            blk['ln1_g'], blk['ln1_b'], blk['qkv_b'], blk['proj_b'],
            blk['ln2_g'], blk['ln2_b'], blk['fc1_b'], blk['fc2_b'],
        ]).reshape(1, _P_LEN) for blk in blocks], axis=0)

    out = pl.pallas_call(
        _mega_body,
        in_specs=[vspec] + [aspec] * (4 * NL) + [vspec] * 14,
        out_specs=vspec,
        out_shape=jax.ShapeDtypeStruct((B, 2), jnp.float32),
        scratch_shapes=[
            pltpu.VMEM((ROWS, D), jnp.float32),
            pltpu.VMEM((D, 3 * D), jnp.float32),
            pltpu.VMEM((D, D), jnp.float32),
            pltpu.VMEM((D, FF), jnp.float32),
            pltpu.VMEM((FF, D), jnp.float32),
            pltpu.VMEM((D, 3 * D), _BF),
            pltpu.VMEM((D, D), _BF),
            pltpu.VMEM((D, FF), _BF),
            pltpu.VMEM((FF, D), _BF),
            pltpu.SemaphoreType.DMA,
            pltpu.SemaphoreType.DMA,
            pltpu.SemaphoreType.DMA,
            pltpu.SemaphoreType.DMA,
        ],
        interpret=interpret,
    )(pr,
      *[blk['qkv_w'] for blk in blocks],
      *[blk['proj_w'] for blk in blocks],
      *[blk['fc1_w'] for blk in blocks],
      *[blk['fc2_w'] for blk in blocks],
      patches.astype(_BF), w_eff.astype(_BF), posb, cls_row,
      _row(params['pre_g']), _row(params['pre_b']),
      _row(params['norm_g']), _row(params['norm_b']),
      _row(params['cls_ln_g']), _row(params['cls_ln_b']),
      params['head1_w'], _row(params['head1_b']),
      params['head2_w'], _row(params['head2_b']))
    return out
